# baseline (speedup 1.0000x reference)
import functools

import jax
import jax.numpy as jnp
from jax import lax
from jax.experimental import pallas as pl
from jax.experimental.pallas import tpu as pltpu
from jax.experimental.pallas import tpu_sc as plsc

NC = 2
NS = 16
L = 16
K = 80
NR = 4
NI = 8


@functools.partial(jax.jit, static_argnames=("B", "N", "D", "CH", "Npad"))
def _sc_aggregate(x_flat, col3, row3, w3, *, B, N, D, CH, Npad):
    npt = Npad // NS

    mesh = plsc.VectorSubcoreMesh(
        core_axis_name="c", subcore_axis_name="s", num_cores=NC)

    scratch = (
        [pltpu.VMEM((K, D), jnp.float32) for _ in range(NR)]
        + [pltpu.VMEM((K,), jnp.int32) for _ in range(NI)]
        + [pltpu.VMEM((K,), jnp.int32) for _ in range(NI)]
        + [pltpu.VMEM((K,), jnp.float32) for _ in range(NI)]
        + [pltpu.VMEM_SHARED((Npad, D), jnp.float32)]
        + [pltpu.SemaphoreType.DMA] * (NI + NR + NR)
    )

    @functools.partial(
        pl.kernel,
        out_type=jax.ShapeDtypeStruct((B * Npad, D), jnp.float32),
        mesh=mesh,
        scratch_types=scratch,
    )
    def sc_kernel(x_hbm, col_hbm, row_hbm, w_hbm, out_hbm, *sc):
        rows = sc[:NR]
        colb = sc[NR:NR + NI]
        rowb = sc[NR + NI:NR + 2 * NI]
        wb = sc[NR + 2 * NI:NR + 3 * NI]
        agg_sh = sc[NR + 3 * NI]
        isems = sc[NR + 3 * NI + 1:NR + 4 * NI + 1]
        gsems = sc[NR + 4 * NI + 1:NR + 4 * NI + 1 + NR]
        ssems = sc[NR + 4 * NI + 1 + NR:]

        c = lax.axis_index("c")
        s = lax.axis_index("s")

        @pl.loop(0, K)
        def _zero_rows(j):
            r0 = rows[0].at[j]
            for g in range(D // L):
                r0[pl.ds(g * L, L)] = jnp.zeros((L,), jnp.float32)

        base = s * npt
        off = 0
        while off < npt:
            sz = min(K, npt - off)
            pltpu.sync_copy(rows[0].at[pl.ds(0, sz)],
                            agg_sh.at[pl.ds(base + off, sz)])
            off += sz
        plsc.subcore_barrier()

        def issue_idx(q, i):
            pltpu.async_copy(col_hbm.at[c, s, q], colb[i], isems[i])
            pltpu.async_copy(row_hbm.at[s, q], rowb[i], isems[i])
            pltpu.async_copy(w_hbm.at[s, q], wb[i], isems[i])

        def wait_idx(i):
            pltpu.make_async_copy(col_hbm.at[c, s, 0], colb[i],
                                  isems[i]).wait()
            pltpu.make_async_copy(row_hbm.at[s, 0], rowb[i],
                                  isems[i]).wait()
            pltpu.make_async_copy(w_hbm.at[s, 0], wb[i], isems[i]).wait()

        def issue_gather(i, r):
            pltpu.async_copy(x_hbm.at[colb[i]], rows[r], gsems[r])

        def wait_gather(i, r):
            pltpu.make_async_copy(x_hbm.at[colb[i]], rows[r],
                                  gsems[r]).wait()

        def issue_scatter(i, r):
            pltpu.async_copy(rows[r], agg_sh.at[rowb[i]], ssems[r],
                             add=True)

        def wait_scatter(i, r):
            pltpu.make_async_copy(rows[r], agg_sh.at[rowb[i]],
                                  ssems[r]).wait()

        def scale(i, r):
            buf = rows[r]
            wref = wb[i]

            @pl.loop(0, K // L)
            def _scale(t):
                w16 = wref[pl.ds(t * L, L)]
                for l in range(L):
                    wv = jnp.full((L,), w16[l], jnp.float32)
                    rj = buf.at[t * L + l]
                    for g in range(D // L):
                        rj[pl.ds(g * L, L)] = rj[pl.ds(g * L, L)] * wv

        for q in range(min(5, CH)):
            issue_idx(q, q)
        for q in range(min(3, CH)):
            wait_idx(q)
            issue_gather(q, q)

        @pl.loop(0, CH, step=NI)
        def _chunk(ch):
            for j in range(NI):
                q = ch + j
                r = j % NR
                r3 = (j + 3) % NR
                i = j
                i3 = (j + 3) % NI
                i5 = (j + 5) % NI
                im1 = (j - 1) % NI
                rm1 = (j - 1) % NR
                wait_gather(i, r)

                @pl.when(q >= 1)
                def _():
                    wait_scatter(im1, rm1)

                @pl.when(q + 3 < CH)
                def _():
                    wait_idx(i3)
                    issue_gather(i3, r3)

                scale(i, r)
                issue_scatter(i, r)

                @pl.when(q + 5 < CH)
                def _():
                    issue_idx(q + 5, i5)

        wait_scatter((CH - 1) % NI, (CH - 1) % NR)
        plsc.subcore_barrier()
        pltpu.sync_copy(agg_sh.at[pl.ds(base, npt)],
                        out_hbm.at[pl.ds(c * Npad + base, npt)])

    return sc_kernel(x_flat, col3, row3, w3)


def _tc_body(a_ref, w_ref, b_ref, o_ref):
    o_ref[...] = jnp.maximum(
        jnp.dot(a_ref[0], w_ref[...], preferred_element_type=jnp.float32)
        + b_ref[...], 0.0)[None]


def _tc_dense(agg_pad, W, b2, *, N, BLK=2000):
    B, Npad, D = agg_pad.shape
    DO = W.shape[1]
    return pl.pallas_call(
        _tc_body,
        grid=(B, N // BLK),
        in_specs=[
            pl.BlockSpec((1, BLK, D), lambda b, i: (b, i, 0)),
            pl.BlockSpec((D, DO), lambda b, i: (0, 0)),
            pl.BlockSpec((1, DO), lambda b, i: (0, 0)),
        ],
        out_specs=pl.BlockSpec((1, BLK, DO), lambda b, i: (b, i, 0)),
        out_shape=jax.ShapeDtypeStruct((B, N, DO), jnp.float32),
    )(agg_pad, W, b2)


def kernel(x, edge_index, edge_weight, W, b):
    B, N, D = x.shape
    E = edge_weight.shape[0]
    row = edge_index[0]
    col = edge_index[1]

    CH = -(-E // (NS * K))
    CH = -(-CH // NI) * NI
    pad = NS * CH * K - E
    row_p = jnp.pad(row, (0, pad))
    w_p = jnp.pad(edge_weight, (0, pad))
    Npad = -(-N // (NS * 8)) * NS * 8
    col3 = (jnp.pad(col, (0, pad))[None, :]
            + (jnp.arange(B, dtype=jnp.int32) * N)[:, None]).reshape(
                B, NS, CH, K)
    row3 = row_p.reshape(NS, CH, K)
    w3 = w_p.reshape(NS, CH, K)

    agg = _sc_aggregate(x.reshape(B * N, D), col3, row3, w3,
                        B=B, N=N, D=D, CH=CH, Npad=Npad)
    out = _tc_dense(agg.reshape(B, Npad, D), W, b.reshape(1, -1), N=N)
    return out


# speedup vs baseline: 1.6961x; 1.6961x over previous
import functools

import jax
import jax.numpy as jnp
from jax import lax
from jax.experimental import pallas as pl
from jax.experimental.pallas import tpu as pltpu
from jax.experimental.pallas import tpu_sc as plsc

NC = 2
NS = 16
L = 16
K = 112
NR = 3
NI = 6


@functools.partial(jax.jit, static_argnames=("B", "N", "D", "CH", "Npad"))
def _sc_aggregate(x_flat, col3, row3, w3, *, B, N, D, CH, Npad):
    npt = Npad // NS

    mesh = plsc.VectorSubcoreMesh(
        core_axis_name="c", subcore_axis_name="s", num_cores=NC)

    scratch = (
        [pltpu.VMEM((K, D), jnp.float32) for _ in range(NR)]
        + [pltpu.VMEM((K,), jnp.int32) for _ in range(NI)]
        + [pltpu.VMEM((K,), jnp.int32) for _ in range(NI)]
        + [pltpu.VMEM((K,), jnp.float32) for _ in range(NI)]
        + [pltpu.VMEM_SHARED((Npad, D), jnp.float32)]
        + [pltpu.SemaphoreType.DMA] * (NI + NR + NR)
    )

    @functools.partial(
        pl.kernel,
        out_type=jax.ShapeDtypeStruct((B * Npad, D), jnp.float32),
        mesh=mesh,
        scratch_types=scratch,
    )
    def sc_kernel(x_hbm, col_hbm, row_hbm, w_hbm, out_hbm, *sc):
        rows = sc[:NR]
        colb = sc[NR:NR + NI]
        rowb = sc[NR + NI:NR + 2 * NI]
        wb = sc[NR + 2 * NI:NR + 3 * NI]
        agg_sh = sc[NR + 3 * NI]
        isems = sc[NR + 3 * NI + 1:NR + 4 * NI + 1]
        gsems = sc[NR + 4 * NI + 1:NR + 4 * NI + 1 + NR]
        ssems = sc[NR + 4 * NI + 1 + NR:]

        c = lax.axis_index("c")
        s = lax.axis_index("s")

        @pl.loop(0, K)
        def _zero_rows(j):
            r0 = rows[0].at[j]
            for g in range(D // L):
                r0[pl.ds(g * L, L)] = jnp.zeros((L,), jnp.float32)

        base = s * npt
        off = 0
        while off < npt:
            sz = min(K, npt - off)
            pltpu.sync_copy(rows[0].at[pl.ds(0, sz)],
                            agg_sh.at[pl.ds(base + off, sz)])
            off += sz
        plsc.subcore_barrier()

        def issue_idx(q, i):
            pltpu.async_copy(col_hbm.at[c, s, q], colb[i], isems[i])
            pltpu.async_copy(row_hbm.at[s, q], rowb[i], isems[i])
            pltpu.async_copy(w_hbm.at[s, q], wb[i], isems[i])

        def wait_idx(i):
            pltpu.make_async_copy(col_hbm.at[c, s, 0], colb[i],
                                  isems[i]).wait()
            pltpu.make_async_copy(row_hbm.at[s, 0], rowb[i],
                                  isems[i]).wait()
            pltpu.make_async_copy(w_hbm.at[s, 0], wb[i], isems[i]).wait()

        def issue_gather(i, r):
            pltpu.async_copy(x_hbm.at[colb[i]], rows[r], gsems[r])

        def wait_gather(i, r):
            pltpu.make_async_copy(x_hbm.at[colb[i]], rows[r],
                                  gsems[r]).wait()

        def issue_scatter(i, r):
            pltpu.async_copy(rows[r], agg_sh.at[rowb[i]], ssems[r],
                             add=True)

        def wait_scatter(i, r):
            pltpu.make_async_copy(rows[r], agg_sh.at[rowb[i]],
                                  ssems[r]).wait()

        def scale(i, r):
            buf = rows[r]
            wref = wb[i]

            @pl.loop(0, K // L)
            def _scale(t):
                w16 = wref[pl.ds(t * L, L)]
                for l in range(L):
                    wv = jnp.full((L,), w16[l], jnp.float32)
                    rj = buf.at[t * L + l]
                    for g in range(D // L):
                        rj[pl.ds(g * L, L)] = rj[pl.ds(g * L, L)] * wv

        for q in range(min(4, CH)):
            issue_idx(q, q)
        for q in range(min(2, CH)):
            wait_idx(q)
            issue_gather(q, q)

        @pl.loop(0, CH, step=NI)
        def _chunk(ch):
            for j in range(NI):
                q = ch + j
                r = j % NR
                r2 = (j + 2) % NR
                i = j
                i2 = (j + 2) % NI
                i4 = (j + 4) % NI
                im1 = (j - 1) % NI
                wait_gather(i, r)

                @pl.when(q >= 1)
                def _():
                    wait_scatter(im1, r2)

                @pl.when(q + 2 < CH)
                def _():
                    wait_idx(i2)
                    issue_gather(i2, r2)

                scale(i, r)
                issue_scatter(i, r)

                @pl.when(q + 4 < CH)
                def _():
                    issue_idx(q + 4, i4)

        wait_scatter((CH - 1) % NI, (CH - 1) % NR)
        plsc.subcore_barrier()
        pltpu.sync_copy(agg_sh.at[pl.ds(base, npt)],
                        out_hbm.at[pl.ds(c * Npad + base, npt)])

    return sc_kernel(x_flat, col3, row3, w3)


def _tc_body(a_ref, w_ref, b_ref, o_ref):
    o_ref[...] = jnp.maximum(
        jnp.dot(a_ref[0], w_ref[...], preferred_element_type=jnp.float32)
        + b_ref[...], 0.0)[None]


def _tc_dense(agg_pad, W, b2, *, N, BLK=2000):
    B, Npad, D = agg_pad.shape
    DO = W.shape[1]
    return pl.pallas_call(
        _tc_body,
        grid=(B, N // BLK),
        in_specs=[
            pl.BlockSpec((1, BLK, D), lambda b, i: (b, i, 0)),
            pl.BlockSpec((D, DO), lambda b, i: (0, 0)),
            pl.BlockSpec((1, DO), lambda b, i: (0, 0)),
        ],
        out_specs=pl.BlockSpec((1, BLK, DO), lambda b, i: (b, i, 0)),
        out_shape=jax.ShapeDtypeStruct((B, N, DO), jnp.float32),
    )(agg_pad, W, b2)


def kernel(x, edge_index, edge_weight, W, b):
    B, N, D = x.shape
    E = edge_weight.shape[0]
    row = edge_index[0]
    col = edge_index[1]

    CH = -(-E // (NS * K))
    CH = -(-CH // NI) * NI
    pad = NS * CH * K - E
    row_p = jnp.pad(row, (0, pad))
    w_p = jnp.pad(edge_weight, (0, pad))
    Npad = -(-N // (NS * 8)) * NS * 8
    col3 = (jnp.pad(col, (0, pad))[None, :]
            + (jnp.arange(B, dtype=jnp.int32) * N)[:, None]).reshape(
                B, NS, CH, K)
    row3 = row_p.reshape(NS, CH, K)
    w3 = w_p.reshape(NS, CH, K)

    agg = _sc_aggregate(x.reshape(B * N, D), col3, row3, w3,
                        B=B, N=N, D=D, CH=CH, Npad=Npad)
    out = _tc_dense(agg.reshape(B, Npad, D), W, b.reshape(1, -1), N=N)
    return out
